# baseline (speedup 1.0000x reference)
import functools
import jax
import jax.numpy as jnp
from jax import lax
from jax.experimental import pallas as pl
from jax.experimental.pallas import tpu as pltpu

N, E, G = 10000, 320000, 256
F, H, C = 79, 8, 128
NP = 10240
FP = 128
FG = 80
BLK = 1024


def _k1_body(x_ref, w_ref, ats_ref, atd_ref, ab_ref, m_ref):
    w3 = w_ref[...].reshape(FP, H, C)
    ws = (w3 * ats_ref[...][None, :, :]).sum(-1)
    wd = (w3 * atd_ref[...][None, :, :]).sum(-1)
    x = x_ref[...]
    a_src = jax.lax.dot(x, ws, preferred_element_type=jnp.float32)
    a_dst = jax.lax.dot(x, wd, preferred_element_type=jnp.float32)
    ab_ref[...] = jnp.concatenate([a_src, a_dst], axis=1)
    m_ref[...] = (jnp.max(a_src) + jnp.max(a_dst)).reshape(1, 1)


def _k1(x_pad, w_pad, att_src, att_dst):
    return pl.pallas_call(
        _k1_body,
        out_shape=[
            jax.ShapeDtypeStruct((NP, 2 * H), jnp.float32),
            jax.ShapeDtypeStruct((1, 1), jnp.float32),
        ],
    )(x_pad, w_pad, att_src, att_dst)


def _k3_body(y_ref, wbd_ref, bias_ref, fcw_ref, batch_ref, fcb_ref, out_ref,
             sums_ref, counts_ref):
    i = pl.program_id(0)

    @pl.when(i == 0)
    def _init():
        sums_ref[...] = jnp.zeros_like(sums_ref)
        counts_ref[...] = jnp.zeros_like(counts_ref)

    y = y_ref[...]
    o = jax.lax.dot(y, wbd_ref[...], preferred_element_type=jnp.float32)
    o = jnp.maximum(o + bias_ref[...], 0.0)
    z = jax.lax.dot(o, fcw_ref[...], preferred_element_type=jnp.float32)

    gids = jax.lax.broadcasted_iota(jnp.int32, (G, BLK), 0)
    oht = (gids == batch_ref[...]).astype(jnp.float32)
    sums_ref[...] += jax.lax.dot(oht, z, preferred_element_type=jnp.float32)
    counts_ref[...] += jnp.sum(oht, axis=1, keepdims=True)

    @pl.when(i == pl.num_programs(0) - 1)
    def _fin():
        out_ref[...] = (sums_ref[...] / jnp.maximum(counts_ref[...], 1.0)
                        + fcb_ref[...])


def _k3(y_pad, w_bd, bias2, fc_w, batch2, fcb2):
    nblk = NP // BLK
    return pl.pallas_call(
        _k3_body,
        grid=(nblk,),
        in_specs=[
            pl.BlockSpec((BLK, H * FG), lambda i: (i, 0)),
            pl.BlockSpec((H * FG, H * C), lambda i: (0, 0)),
            pl.BlockSpec((1, H * C), lambda i: (0, 0)),
            pl.BlockSpec((H * C, 1), lambda i: (0, 0)),
            pl.BlockSpec((1, BLK), lambda i: (0, i)),
            pl.BlockSpec((1, 1), lambda i: (0, 0)),
        ],
        out_specs=pl.BlockSpec((G, 1), lambda i: (0, 0)),
        out_shape=jax.ShapeDtypeStruct((G, 1), jnp.float32),
        scratch_shapes=[
            pltpu.VMEM((G, 1), jnp.float32),
            pltpu.VMEM((G, 1), jnp.float32),
        ],
    )(y_pad, w_bd, bias2, fc_w, batch2, fcb2)


def _edge_phase(ab, m, x_g, edge_index):
    src = edge_index[0]
    dst = edge_index[1]
    alpha = ab[src, 0:H] + ab[dst, H:2 * H]
    alpha = jnp.where(alpha >= 0, alpha, 0.2 * alpha)
    eal = jnp.exp(alpha - m[0, 0])
    denom = jax.ops.segment_sum(eal, dst, num_segments=N)
    coef = eal / (denom[dst] + 1e-16)
    msg = (coef[:, :, None] * x_g[src][:, None, :]).reshape(E, H * FG)
    return jax.ops.segment_sum(msg, dst, num_segments=N)


def kernel(x, edge_index, batch, W, att_src, att_dst, bias_conv, fc_w, fc_b):
    x_pad = jnp.zeros((NP, FP), jnp.float32).at[:N, :F].set(x)
    w_pad = jnp.zeros((FP, H * C), jnp.float32).at[:F].set(W)
    ab, m = _k1(x_pad, w_pad, att_src[0], att_dst[0])

    x_g = x_pad[:N, :FG]
    y = _edge_phase(ab[:N], m, x_g, edge_index)
    y_pad = jnp.zeros((NP, H * FG), jnp.float32).at[:N].set(y)

    w_bd = jnp.zeros((H, FG, H, C), jnp.float32)
    for h in range(H):
        w_bd = w_bd.at[h, :F, h, :].set(W[:, h * C:(h + 1) * C])
    w_bd = w_bd.reshape(H * FG, H * C)

    batch2 = jnp.full((1, NP), G, jnp.int32).at[0, :N].set(batch.astype(jnp.int32))
    out = _k3(y_pad, w_bd, bias_conv.reshape(1, H * C), fc_w, batch2,
              fc_b.reshape(1, 1))
    return out


# speedup vs baseline: 101.1642x; 101.1642x over previous
import functools
import jax
import jax.numpy as jnp
from jax import lax
from jax.experimental import pallas as pl
from jax.experimental.pallas import tpu as pltpu

N, E, G = 10000, 320000, 256
F, H, C = 79, 8, 128
NP = 10240
FP = 128
FG = 80
BLK = 1024


def _k1_body(x_ref, w_ref, ats_ref, atd_ref, ab_ref, m_ref):
    w3 = w_ref[...].reshape(FP, H, C)
    ws = (w3 * ats_ref[...][None, :, :]).sum(-1)
    wd = (w3 * atd_ref[...][None, :, :]).sum(-1)
    x = x_ref[...]
    a_src = jax.lax.dot(x, ws, preferred_element_type=jnp.float32)
    a_dst = jax.lax.dot(x, wd, preferred_element_type=jnp.float32)
    ab_ref[...] = jnp.concatenate([a_src, a_dst], axis=1)
    m_ref[...] = (jnp.max(a_src) + jnp.max(a_dst)).reshape(1, 1)


def _k1(x_pad, w_pad, att_src, att_dst):
    return pl.pallas_call(
        _k1_body,
        out_shape=[
            jax.ShapeDtypeStruct((NP, 2 * H), jnp.float32),
            jax.ShapeDtypeStruct((1, 1), jnp.float32),
        ],
    )(x_pad, w_pad, att_src, att_dst)


def _k3_body(y_ref, wbd_ref, bias_ref, fcw_ref, batch_ref, fcb_ref, out_ref,
             sums_ref, counts_ref):
    i = pl.program_id(0)

    @pl.when(i == 0)
    def _init():
        sums_ref[...] = jnp.zeros_like(sums_ref)
        counts_ref[...] = jnp.zeros_like(counts_ref)

    y = y_ref[...]
    o = jax.lax.dot(y, wbd_ref[...], preferred_element_type=jnp.float32)
    o = jnp.maximum(o + bias_ref[...], 0.0)
    z = jax.lax.dot(o, fcw_ref[...], preferred_element_type=jnp.float32)

    gids = jax.lax.broadcasted_iota(jnp.int32, (G, BLK), 0)
    oht = (gids == batch_ref[...]).astype(jnp.float32)
    sums_ref[...] += jax.lax.dot(oht, z, preferred_element_type=jnp.float32)
    counts_ref[...] += jnp.sum(oht, axis=1, keepdims=True)

    @pl.when(i == pl.num_programs(0) - 1)
    def _fin():
        out_ref[...] = (sums_ref[...] / jnp.maximum(counts_ref[...], 1.0)
                        + fcb_ref[...])


def _k3(y_pad, w_bd, bias2, fc_w, batch2, fcb2):
    nblk = NP // BLK
    return pl.pallas_call(
        _k3_body,
        grid=(nblk,),
        in_specs=[
            pl.BlockSpec((BLK, H * FG), lambda i: (i, 0)),
            pl.BlockSpec((H * FG, H * C), lambda i: (0, 0)),
            pl.BlockSpec((1, H * C), lambda i: (0, 0)),
            pl.BlockSpec((H * C, 1), lambda i: (0, 0)),
            pl.BlockSpec((1, BLK), lambda i: (0, i)),
            pl.BlockSpec((1, 1), lambda i: (0, 0)),
        ],
        out_specs=pl.BlockSpec((G, 1), lambda i: (0, 0)),
        out_shape=jax.ShapeDtypeStruct((G, 1), jnp.float32),
        scratch_shapes=[
            pltpu.VMEM((G, 1), jnp.float32),
            pltpu.VMEM((G, 1), jnp.float32),
        ],
    )(y_pad, w_bd, bias2, fc_w, batch2, fcb2)


from jax.experimental.pallas import tpu_sc as plsc

NC, NS, L = 2, 16, 16
NW = NC * NS
BW = 128
NB = NP // BW
ROUNDS = (NB + NW - 1) // NW
CAP = 6400
ECHUNK = 1280
GB = 64


def _vgather(vec, idx):
    dnums = jax.lax.GatherDimensionNumbers(
        offset_dims=(), collapsed_slice_dims=(0,), start_index_map=(0,))
    return jax.lax.gather(vec, idx[:, None], dnums, (1,),
                          mode=jax.lax.GatherScatterMode.PROMISE_IN_BOUNDS)


def _k2_body(ab_hbm, x_hbm, src_hbm, dst_hbm, mv_hbm, y_hbm,
             slist, dlist, absrc, abdst, xbuf, denom, inv, ylocal,
             srcc, dstc, mvec, sem1, sem2, sem3):
    wid = jax.lax.axis_index("s") * NC + jax.lax.axis_index("c")
    iota = jax.lax.iota(jnp.int32, L)
    lane8 = iota < 8
    rot8 = 8 + jnp.bitwise_and(iota, 7)
    zeros16 = jnp.zeros((L,), jnp.float32)

    pltpu.sync_copy(mv_hbm, mvec)
    mval = mvec[...]

    for r in range(ROUNDS):
        bucket = r * NW + wid
        lo = bucket * BW

        @pl.when(bucket < NB)
        def _round():
            def _zy(i, _):
                ylocal[pl.ds(i * L, L)] = zeros16
                return None
            jax.lax.fori_loop(0, BW * H * FG // L, _zy, None)

            def _zd(i, _):
                denom[pl.ds(i * L, L)] = zeros16
                return None
            jax.lax.fori_loop(0, (BW * H + L) // L, _zd, None)

            def _zi(i, _):
                slist[pl.ds(i * L, L)] = jnp.zeros((L,), jnp.int32)
                dlist[pl.ds(i * L, L)] = jnp.zeros((L,), jnp.int32)
                return None
            jax.lax.fori_loop(0, (CAP + GB) // L, _zi, None)

            def _chunk(c, cnt):
                pltpu.sync_copy(src_hbm.at[pl.ds(c * ECHUNK, ECHUNK)], srcc)
                pltpu.sync_copy(dst_hbm.at[pl.ds(c * ECHUNK, ECHUNK)], dstc)

                def _grp(j, cnt):
                    s16 = srcc[pl.ds(j * L, L)]
                    d16 = dstc[pl.ds(j * L, L)]
                    msk = (d16 >= lo) & (d16 < lo + BW)
                    run = plsc.cumsum(jnp.where(msk, 1, 0))
                    pos = cnt + run - 1
                    plsc.store_scatter(slist, [pos], s16, mask=msk)
                    plsc.store_scatter(dlist, [pos], d16, mask=msk)
                    return jnp.minimum(cnt + run[L - 1], CAP)

                return jax.lax.fori_loop(0, ECHUNK // L, _grp, cnt)

            cnt = jax.lax.fori_loop(0, E // ECHUNK, _chunk, 0)
            nblk = (cnt + GB - 1) // GB

            def _s1(b, _):
                idx_s = slist.at[pl.ds(b * GB, GB)]
                idx_d = dlist.at[pl.ds(b * GB, GB)]
                c1 = pltpu.async_copy(ab_hbm.at[idx_s], absrc, sem1)
                c2 = pltpu.async_copy(ab_hbm.at[idx_d], abdst, sem2)
                c1.wait()
                c2.wait()

                def _edge(j, v):
                    @pl.when(b * GB + j < cnt)
                    def _():
                        rs = absrc[j, :]
                        rd = _vgather(abdst[j, :], rot8)
                        al = rs + rd
                        al = jnp.maximum(al, 0.2 * al)
                        eal = jnp.exp(al - mval)
                        dstloc = dlist[pl.ds(b * GB + j, L)][0] - lo
                        plsc.addupdate_scatter(
                            denom, [dstloc * 8 + iota], eal, mask=lane8)
                    return v
                jax.lax.fori_loop(0, GB, _edge, None)
                return None
            jax.lax.fori_loop(0, nblk, _s1, None)

            def _rcp(i, _):
                inv[pl.ds(i * L, L)] = 1.0 / (denom[pl.ds(i * L, L)] + 1e-16)
                return None
            jax.lax.fori_loop(0, (BW * H + L) // L, _rcp, None)

            def _s2(b, _):
                idx_s = slist.at[pl.ds(b * GB, GB)]
                idx_d = dlist.at[pl.ds(b * GB, GB)]
                c1 = pltpu.async_copy(ab_hbm.at[idx_s], absrc, sem1)
                c2 = pltpu.async_copy(ab_hbm.at[idx_d], abdst, sem2)
                c3 = pltpu.async_copy(x_hbm.at[idx_s], xbuf, sem3)
                c1.wait()
                c2.wait()
                c3.wait()

                def _edge(j, v):
                    @pl.when(b * GB + j < cnt)
                    def _():
                        rs = absrc[j, :]
                        rd = _vgather(abdst[j, :], rot8)
                        al = rs + rd
                        al = jnp.maximum(al, 0.2 * al)
                        eal = jnp.exp(al - mval)
                        dstloc = dlist[pl.ds(b * GB + j, L)][0] - lo
                        invrow = inv[pl.ds(dstloc * 8, L)]
                        coef = eal * invrow
                        xs = [xbuf[j, pl.ds(k * L, L)] for k in range(FG // L)]
                        ybase = dstloc * (H * FG)
                        for h in range(H):
                            ch = _vgather(coef, jnp.full((L,), h, jnp.int32))
                            for k in range(FG // L):
                                plsc.addupdate(
                                    ylocal.at[pl.ds(ybase + h * FG + k * L, L)],
                                    ch * xs[k])
                    return v
                jax.lax.fori_loop(0, GB, _edge, None)
                return None
            jax.lax.fori_loop(0, nblk, _s2, None)

            pltpu.sync_copy(ylocal, y_hbm.at[pl.ds(lo * (H * FG), BW * H * FG)])


def _k2(ab, x_g, src, dst, mvec16):
    mesh = plsc.VectorSubcoreMesh(core_axis_name="c", subcore_axis_name="s")
    f = pl.kernel(
        _k2_body,
        out_type=jax.ShapeDtypeStruct((NP * H * FG,), jnp.float32),
        mesh=mesh,
        compiler_params=pltpu.CompilerParams(needs_layout_passes=False,
                                             use_tc_tiling_on_sc=False),
        scratch_types=[
            pltpu.VMEM((CAP + GB,), jnp.int32),
            pltpu.VMEM((CAP + GB,), jnp.int32),
            pltpu.VMEM((GB, 2 * H), jnp.float32),
            pltpu.VMEM((GB, 2 * H), jnp.float32),
            pltpu.VMEM((GB, FG), jnp.float32),
            pltpu.VMEM((BW * H + L,), jnp.float32),
            pltpu.VMEM((BW * H + L,), jnp.float32),
            pltpu.VMEM((BW * H * FG,), jnp.float32),
            pltpu.VMEM((ECHUNK,), jnp.int32),
            pltpu.VMEM((ECHUNK,), jnp.int32),
            pltpu.VMEM((L,), jnp.float32),
            pltpu.SemaphoreType.DMA,
            pltpu.SemaphoreType.DMA,
            pltpu.SemaphoreType.DMA,
        ],
    )
    return f(ab, x_g, src, dst, mvec16)


def kernel(x, edge_index, batch, W, att_src, att_dst, bias_conv, fc_w, fc_b):
    x_pad = jnp.zeros((NP, FP), jnp.float32).at[:N, :F].set(x)
    w_pad = jnp.zeros((FP, H * C), jnp.float32).at[:F].set(W)
    ab, m = _k1(x_pad, w_pad, att_src[0], att_dst[0])

    x_g = x_pad[:N, :FG]
    mvec16 = jnp.broadcast_to(m.reshape(()), (16,))
    y_flat = _k2(ab[:N], x_g, edge_index[0], edge_index[1], mvec16)
    y_pad = y_flat.reshape(NP, H * FG)

    w_bd = jnp.zeros((H, FG, H, C), jnp.float32)
    for h in range(H):
        w_bd = w_bd.at[h, :F, h, :].set(W[:, h * C:(h + 1) * C])
    w_bd = w_bd.reshape(H * FG, H * C)

    batch2 = jnp.full((1, NP), G, jnp.int32).at[0, :N].set(batch.astype(jnp.int32))
    out = _k3(y_pad, w_bd, bias_conv.reshape(1, H * C), fc_w, batch2,
              fc_b.reshape(1, 1))
    return out


# speedup vs baseline: 170.7470x; 1.6878x over previous
import functools
import jax
import jax.numpy as jnp
from jax import lax
from jax.experimental import pallas as pl
from jax.experimental.pallas import tpu as pltpu

N, E, G = 10000, 320000, 256
F, H, C = 79, 8, 128
NP = 10240
FP = 128
FG = 80
BLK = 1024


def _k1_body(x_ref, w_ref, ats_ref, atd_ref, ab_ref, m_ref):
    w3 = w_ref[...].reshape(FP, H, C)
    ws = (w3 * ats_ref[...][None, :, :]).sum(-1)
    wd = (w3 * atd_ref[...][None, :, :]).sum(-1)
    x = x_ref[...]
    a_src = jax.lax.dot(x, ws, preferred_element_type=jnp.float32)
    a_dst = jax.lax.dot(x, wd, preferred_element_type=jnp.float32)
    ab_ref[...] = jnp.concatenate([a_src, a_dst], axis=1)
    m_ref[...] = (jnp.max(a_src) + jnp.max(a_dst)).reshape(1, 1)


def _k1(x_pad, w_pad, att_src, att_dst):
    return pl.pallas_call(
        _k1_body,
        out_shape=[
            jax.ShapeDtypeStruct((NP, 2 * H), jnp.float32),
            jax.ShapeDtypeStruct((1, 1), jnp.float32),
        ],
    )(x_pad, w_pad, att_src, att_dst)


def _k3_body(y_ref, wbd_ref, bias_ref, fcw_ref, batch_ref, fcb_ref, out_ref,
             sums_ref, counts_ref):
    i = pl.program_id(0)

    @pl.when(i == 0)
    def _init():
        sums_ref[...] = jnp.zeros_like(sums_ref)
        counts_ref[...] = jnp.zeros_like(counts_ref)

    y = y_ref[...]
    o = jax.lax.dot(y, wbd_ref[...], preferred_element_type=jnp.float32)
    o = jnp.maximum(o + bias_ref[...], 0.0)
    z = jax.lax.dot(o, fcw_ref[...], preferred_element_type=jnp.float32)

    gids = jax.lax.broadcasted_iota(jnp.int32, (G, BLK), 0)
    oht = (gids == batch_ref[...]).astype(jnp.float32)
    sums_ref[...] += jax.lax.dot(oht, z, preferred_element_type=jnp.float32)
    counts_ref[...] += jnp.sum(oht, axis=1, keepdims=True)

    @pl.when(i == pl.num_programs(0) - 1)
    def _fin():
        out_ref[...] = (sums_ref[...] / jnp.maximum(counts_ref[...], 1.0)
                        + fcb_ref[...])


def _k3(y_pad, w_bd, bias2, fc_w, batch2, fcb2):
    nblk = NP // BLK
    return pl.pallas_call(
        _k3_body,
        grid=(nblk,),
        in_specs=[
            pl.BlockSpec((BLK, H * FG), lambda i: (i, 0)),
            pl.BlockSpec((H * FG, H * C), lambda i: (0, 0)),
            pl.BlockSpec((1, H * C), lambda i: (0, 0)),
            pl.BlockSpec((H * C, 1), lambda i: (0, 0)),
            pl.BlockSpec((1, BLK), lambda i: (0, i)),
            pl.BlockSpec((1, 1), lambda i: (0, 0)),
        ],
        out_specs=pl.BlockSpec((G, 1), lambda i: (0, 0)),
        out_shape=jax.ShapeDtypeStruct((G, 1), jnp.float32),
        scratch_shapes=[
            pltpu.VMEM((G, 1), jnp.float32),
            pltpu.VMEM((G, 1), jnp.float32),
        ],
    )(y_pad, w_bd, bias2, fc_w, batch2, fcb2)


from jax.experimental.pallas import tpu_sc as plsc

NC, NS, L = 2, 16, 16
NW = NC * NS
BW = 64
NB = NP // BW
ROUNDS = NB // NW
CCAP = 11264
BCAP = 2432
ECHUNK = 1280
GB = 64


def _vgather(vec, idx):
    dnums = jax.lax.GatherDimensionNumbers(
        offset_dims=(), collapsed_slice_dims=(0,), start_index_map=(0,))
    return jax.lax.gather(vec, idx[:, None], dnums, (1,),
                          mode=jax.lax.GatherScatterMode.PROMISE_IN_BOUNDS)


def _k2_body(ab_hbm, x_hbm, src_hbm, dst_hbm, mv_hbm, y_hbm,
             csl, cdl, slist, dlist, absrc, abdst, xbuf, ealbuf,
             denom, inv, ylocal, srcc, dstc, mvec, sem1, sem2, sem3):
    wid = jax.lax.axis_index("s") * NC + jax.lax.axis_index("c")
    iota = jax.lax.iota(jnp.int32, L)
    lane8 = iota < 8
    rot8 = 8 + jnp.bitwise_and(iota, 7)
    zeros16 = jnp.zeros((L,), jnp.float32)

    pltpu.sync_copy(mv_hbm, mvec)
    mval = mvec[...]

    def _chunk(c, cnt):
        pltpu.sync_copy(src_hbm.at[pl.ds(c * ECHUNK, ECHUNK)], srcc)
        pltpu.sync_copy(dst_hbm.at[pl.ds(c * ECHUNK, ECHUNK)], dstc)

        def _grp(j, cnt):
            s16 = srcc[pl.ds(j * L, L)]
            d16 = dstc[pl.ds(j * L, L)]
            bkt = jax.lax.shift_right_logical(d16, 6)
            msk = jnp.bitwise_and(bkt, NW - 1) == wid
            run = plsc.cumsum(jnp.where(msk, 1, 0))
            pos = cnt + run - 1
            plsc.store_scatter(csl, [pos], s16, mask=msk)
            plsc.store_scatter(cdl, [pos], d16, mask=msk)
            return jnp.minimum(cnt + run[L - 1], CCAP)

        return jax.lax.fori_loop(0, ECHUNK // L, _grp, cnt)

    cnt_all = jax.lax.fori_loop(0, E // ECHUNK, _chunk, 0)
    nit_all = (cnt_all + L - 1) // L

    for r in range(ROUNDS):
        bucket = r * NW + wid
        lo = bucket * BW

        def _zy(i, _):
            ylocal[pl.ds(i * L, L)] = zeros16
            return None
        jax.lax.fori_loop(0, BW * H * FG // L, _zy, None)

        def _zd(i, _):
            denom[pl.ds(i * L, L)] = zeros16
            return None
        jax.lax.fori_loop(0, (BW * H + L) // L, _zd, None)

        def _zi(i, _):
            slist[pl.ds(i * L, L)] = jnp.zeros((L,), jnp.int32)
            dlist[pl.ds(i * L, L)] = jnp.zeros((L,), jnp.int32)
            return None
        jax.lax.fori_loop(0, (BCAP + GB) // L, _zi, None)

        def _part(i, cnt):
            s16 = csl[pl.ds(i * L, L)]
            d16 = cdl[pl.ds(i * L, L)]
            valid = (i * L + iota) < cnt_all
            msk = valid & (d16 >= lo) & (d16 < lo + BW)
            run = plsc.cumsum(jnp.where(msk, 1, 0))
            pos = cnt + run - 1
            plsc.store_scatter(slist, [pos], s16, mask=msk)
            plsc.store_scatter(dlist, [pos], d16, mask=msk)
            return jnp.minimum(cnt + run[L - 1], BCAP)

        cnt = jax.lax.fori_loop(0, nit_all, _part, 0)
        nblk = (cnt + GB - 1) // GB

        def _s1(b, _):
            idx_s = slist.at[pl.ds(b * GB, GB)]
            idx_d = dlist.at[pl.ds(b * GB, GB)]
            c1 = pltpu.async_copy(ab_hbm.at[idx_s], absrc, sem1)
            c2 = pltpu.async_copy(ab_hbm.at[idx_d], abdst, sem2)
            c1.wait()
            c2.wait()

            def _edge(j, v):
                @pl.when(b * GB + j < cnt)
                def _():
                    rs = absrc[j, :]
                    rd = _vgather(abdst[j, :], rot8)
                    al = rs + rd
                    al = jnp.maximum(al, 0.2 * al)
                    eal = jnp.exp(al - mval)
                    epos = b * GB + j
                    dstloc = dlist[pl.ds(epos, L)][0] - lo
                    plsc.addupdate_scatter(
                        denom, [dstloc * 8 + iota], eal, mask=lane8)
                    plsc.store_scatter(ealbuf, [epos * 8 + iota], eal,
                                       mask=lane8)
                return v
            jax.lax.fori_loop(0, GB, _edge, None)
            return None
        jax.lax.fori_loop(0, nblk, _s1, None)

        def _rcp(i, _):
            inv[pl.ds(i * L, L)] = 1.0 / (denom[pl.ds(i * L, L)] + 1e-16)
            return None
        jax.lax.fori_loop(0, (BW * H + L) // L, _rcp, None)

        def _s2(b, _):
            idx_s = slist.at[pl.ds(b * GB, GB)]
            c3 = pltpu.async_copy(x_hbm.at[idx_s], xbuf, sem3)
            c3.wait()

            def _edge(j, v):
                @pl.when(b * GB + j < cnt)
                def _():
                    epos = b * GB + j
                    eal = ealbuf[pl.ds(epos * 8, L)]
                    dstloc = dlist[pl.ds(epos, L)][0] - lo
                    invrow = inv[pl.ds(dstloc * 8, L)]
                    coef = eal * invrow
                    xs = [xbuf[j, pl.ds(k * L, L)] for k in range(FG // L)]
                    ybase = dstloc * (H * FG)
                    for h in range(H):
                        ch = _vgather(coef, jnp.full((L,), h, jnp.int32))
                        for k in range(FG // L):
                            plsc.addupdate(
                                ylocal.at[pl.ds(ybase + h * FG + k * L, L)],
                                ch * xs[k])
                return v
            jax.lax.fori_loop(0, GB, _edge, None)
            return None
        jax.lax.fori_loop(0, nblk, _s2, None)

        pltpu.sync_copy(ylocal, y_hbm.at[pl.ds(lo * (H * FG), BW * H * FG)])


def _k2(ab, x_g, src, dst, mvec16):
    mesh = plsc.VectorSubcoreMesh(core_axis_name="c", subcore_axis_name="s")
    f = pl.kernel(
        _k2_body,
        out_type=jax.ShapeDtypeStruct((NP * H * FG,), jnp.float32),
        mesh=mesh,
        compiler_params=pltpu.CompilerParams(needs_layout_passes=False,
                                             use_tc_tiling_on_sc=False),
        scratch_types=[
            pltpu.VMEM((CCAP + L,), jnp.int32),
            pltpu.VMEM((CCAP + L,), jnp.int32),
            pltpu.VMEM((BCAP + GB,), jnp.int32),
            pltpu.VMEM((BCAP + GB,), jnp.int32),
            pltpu.VMEM((GB, 2 * H), jnp.float32),
            pltpu.VMEM((GB, 2 * H), jnp.float32),
            pltpu.VMEM((GB, FG), jnp.float32),
            pltpu.VMEM((BCAP * 8 + L,), jnp.float32),
            pltpu.VMEM((BW * H + L,), jnp.float32),
            pltpu.VMEM((BW * H + L,), jnp.float32),
            pltpu.VMEM((BW * H * FG,), jnp.float32),
            pltpu.VMEM((ECHUNK,), jnp.int32),
            pltpu.VMEM((ECHUNK,), jnp.int32),
            pltpu.VMEM((L,), jnp.float32),
            pltpu.SemaphoreType.DMA,
            pltpu.SemaphoreType.DMA,
            pltpu.SemaphoreType.DMA,
        ],
    )
    return f(ab, x_g, src, dst, mvec16)


def kernel(x, edge_index, batch, W, att_src, att_dst, bias_conv, fc_w, fc_b):
    x_pad = jnp.zeros((NP, FP), jnp.float32).at[:N, :F].set(x)
    w_pad = jnp.zeros((FP, H * C), jnp.float32).at[:F].set(W)
    ab, m = _k1(x_pad, w_pad, att_src[0], att_dst[0])

    x_g = x_pad[:N, :FG]
    mvec16 = jnp.broadcast_to(m.reshape(()), (16,))
    y_flat = _k2(ab[:N], x_g, edge_index[0], edge_index[1], mvec16)
    y_pad = y_flat.reshape(NP, H * FG)

    w_bd = jnp.zeros((H, FG, H, C), jnp.float32)
    for h in range(H):
        w_bd = w_bd.at[h, :F, h, :].set(W[:, h * C:(h + 1) * C])
    w_bd = w_bd.reshape(H * FG, H * C)

    batch2 = jnp.full((1, NP), G, jnp.int32).at[0, :N].set(batch.astype(jnp.int32))
    out = _k3(y_pad, w_bd, bias_conv.reshape(1, H * C), fc_w, batch2,
              fc_b.reshape(1, 1))
    return out


# speedup vs baseline: 203.6844x; 1.1929x over previous
import functools
import jax
import jax.numpy as jnp
from jax import lax
from jax.experimental import pallas as pl
from jax.experimental.pallas import tpu as pltpu

N, E, G = 10000, 320000, 256
F, H, C = 79, 8, 128
NP = 10240
FP = 128
FG = 80
BLK = 1024


def _k1_body(x_ref, w_ref, ats_ref, atd_ref, ab_ref, m_ref):
    w3 = w_ref[...].reshape(FP, H, C)
    ws = (w3 * ats_ref[...][None, :, :]).sum(-1)
    wd = (w3 * atd_ref[...][None, :, :]).sum(-1)
    x = x_ref[...]
    a_src = jax.lax.dot(x, ws, preferred_element_type=jnp.float32)
    a_dst = jax.lax.dot(x, wd, preferred_element_type=jnp.float32)
    ab_ref[...] = jnp.concatenate([a_src, a_dst], axis=1)
    m_ref[...] = (jnp.max(a_src) + jnp.max(a_dst)).reshape(1, 1)


def _k1(x_pad, w_pad, att_src, att_dst):
    return pl.pallas_call(
        _k1_body,
        out_shape=[
            jax.ShapeDtypeStruct((NP, 2 * H), jnp.float32),
            jax.ShapeDtypeStruct((1, 1), jnp.float32),
        ],
    )(x_pad, w_pad, att_src, att_dst)


def _k3_body(y_ref, wbd_ref, bias_ref, fcw_ref, batch_ref, fcb_ref, out_ref,
             sums_ref, counts_ref):
    i = pl.program_id(0)

    @pl.when(i == 0)
    def _init():
        sums_ref[...] = jnp.zeros_like(sums_ref)
        counts_ref[...] = jnp.zeros_like(counts_ref)

    y = y_ref[...]
    o = jax.lax.dot(y, wbd_ref[...], preferred_element_type=jnp.float32)
    o = jnp.maximum(o + bias_ref[...], 0.0)
    z = jax.lax.dot(o, fcw_ref[...], preferred_element_type=jnp.float32)

    gids = jax.lax.broadcasted_iota(jnp.int32, (G, BLK), 0)
    oht = (gids == batch_ref[...]).astype(jnp.float32)
    sums_ref[...] += jax.lax.dot(oht, z, preferred_element_type=jnp.float32)
    counts_ref[...] += jnp.sum(oht, axis=1, keepdims=True)

    @pl.when(i == pl.num_programs(0) - 1)
    def _fin():
        out_ref[...] = (sums_ref[...] / jnp.maximum(counts_ref[...], 1.0)
                        + fcb_ref[...])


def _k3(y_pad, w_bd, bias2, fc_w, batch2, fcb2):
    nblk = NP // BLK
    return pl.pallas_call(
        _k3_body,
        grid=(nblk,),
        in_specs=[
            pl.BlockSpec((BLK, H * FG), lambda i: (i, 0)),
            pl.BlockSpec((H * FG, H * C), lambda i: (0, 0)),
            pl.BlockSpec((1, H * C), lambda i: (0, 0)),
            pl.BlockSpec((H * C, 1), lambda i: (0, 0)),
            pl.BlockSpec((1, BLK), lambda i: (0, i)),
            pl.BlockSpec((1, 1), lambda i: (0, 0)),
        ],
        out_specs=pl.BlockSpec((G, 1), lambda i: (0, 0)),
        out_shape=jax.ShapeDtypeStruct((G, 1), jnp.float32),
        scratch_shapes=[
            pltpu.VMEM((G, 1), jnp.float32),
            pltpu.VMEM((G, 1), jnp.float32),
        ],
    )(y_pad, w_bd, bias2, fc_w, batch2, fcb2)


from jax.experimental.pallas import tpu_sc as plsc

NC, NS, L = 2, 16, 16
NW = NC * NS
BW = 64
NB = NP // BW
ROUNDS = NB // NW
CCAP = 11264
BCAP = 2432
ECHUNK = 1280
GB = 64


def _vgather(vec, idx):
    dnums = jax.lax.GatherDimensionNumbers(
        offset_dims=(), collapsed_slice_dims=(0,), start_index_map=(0,))
    return jax.lax.gather(vec, idx[:, None], dnums, (1,),
                          mode=jax.lax.GatherScatterMode.PROMISE_IN_BOUNDS)


def _k2_body(ab_hbm, x_hbm, src_hbm, dst_hbm, mv_hbm, y_hbm,
             csl, cdl, slist, dlist, absrc, abdst, xbuf,
             denom, inv, ylocal, srcc, dstc, mvec, sem1, sem2, sem3):
    wid = jax.lax.axis_index("s") * NC + jax.lax.axis_index("c")
    iota = jax.lax.iota(jnp.int32, L)
    lane8 = iota < 8
    rot8 = 8 + jnp.bitwise_and(iota, 7)
    zeros16 = jnp.zeros((L,), jnp.float32)

    pltpu.sync_copy(mv_hbm, mvec)
    mval = mvec[...]

    def _chunk(c, cnt):
        pltpu.sync_copy(src_hbm.at[pl.ds(c * ECHUNK, ECHUNK)], srcc)
        pltpu.sync_copy(dst_hbm.at[pl.ds(c * ECHUNK, ECHUNK)], dstc)

        def _grp(j, cnt):
            s16 = srcc[pl.ds(j * L, L)]
            d16 = dstc[pl.ds(j * L, L)]
            bkt = jax.lax.shift_right_logical(d16, 6)
            msk = jnp.bitwise_and(bkt, NW - 1) == wid
            run = plsc.cumsum(jnp.where(msk, 1, 0))
            pos = cnt + run - 1
            plsc.store_scatter(csl, [pos], s16, mask=msk)
            plsc.store_scatter(cdl, [pos], d16, mask=msk)
            return jnp.minimum(cnt + run[L - 1], CCAP)

        return jax.lax.fori_loop(0, ECHUNK // L, _grp, cnt)

    cnt_all = jax.lax.fori_loop(0, E // ECHUNK, _chunk, 0)
    nit_all = (cnt_all + L - 1) // L

    for r in range(ROUNDS):
        bucket = r * NW + wid
        lo = bucket * BW

        def _zy(i, _):
            ylocal[pl.ds(i * L, L)] = zeros16
            return None
        jax.lax.fori_loop(0, BW * H * FG // L, _zy, None)

        def _zd(i, _):
            denom[pl.ds(i * L, L)] = zeros16
            return None
        jax.lax.fori_loop(0, (BW * H + L) // L, _zd, None)

        def _zi(i, _):
            slist[pl.ds(i * L, L)] = jnp.zeros((L,), jnp.int32)
            dlist[pl.ds(i * L, L)] = jnp.zeros((L,), jnp.int32)
            return None
        jax.lax.fori_loop(0, (BCAP + GB) // L, _zi, None)

        def _part(i, cnt):
            s16 = csl[pl.ds(i * L, L)]
            d16 = cdl[pl.ds(i * L, L)]
            valid = (i * L + iota) < cnt_all
            msk = valid & (d16 >= lo) & (d16 < lo + BW)
            run = plsc.cumsum(jnp.where(msk, 1, 0))
            pos = cnt + run - 1
            plsc.store_scatter(slist, [pos], s16, mask=msk)
            plsc.store_scatter(dlist, [pos], d16, mask=msk)
            return jnp.minimum(cnt + run[L - 1], BCAP)

        cnt = jax.lax.fori_loop(0, nit_all, _part, 0)
        nblk = (cnt + GB - 1) // GB

        def _s1(b, _):
            idx_s = slist.at[pl.ds(b * GB, GB)]
            idx_d = dlist.at[pl.ds(b * GB, GB)]
            c1 = pltpu.async_copy(ab_hbm.at[idx_s], absrc, sem1)
            c2 = pltpu.async_copy(ab_hbm.at[idx_d], abdst, sem2)
            c3 = pltpu.async_copy(x_hbm.at[idx_s], xbuf, sem3)
            c1.wait()
            c2.wait()
            c3.wait()

            def _edge(j, v):
                @pl.when(b * GB + j < cnt)
                def _():
                    rs = absrc[j, :]
                    rd = _vgather(abdst[j, :], rot8)
                    al = rs + rd
                    al = jnp.maximum(al, 0.2 * al)
                    eal = jnp.exp(al - mval)
                    dstloc = dlist[pl.ds(b * GB + j, L)][0] - lo
                    plsc.addupdate_scatter(
                        denom, [dstloc * 8 + iota], eal, mask=lane8)
                    xs = [xbuf[j, pl.ds(k * L, L)] for k in range(FG // L)]
                    ybase = dstloc * (H * FG)
                    for h in range(H):
                        ch = _vgather(eal, jnp.full((L,), h, jnp.int32))
                        for k in range(FG // L):
                            plsc.addupdate(
                                ylocal.at[pl.ds(ybase + h * FG + k * L, L)],
                                ch * xs[k])
                return v
            jax.lax.fori_loop(0, GB, _edge, None)
            return None
        jax.lax.fori_loop(0, nblk, _s1, None)

        def _rcp(i, _):
            inv[pl.ds(i * L, L)] = 1.0 / (denom[pl.ds(i * L, L)] + 1e-16)
            return None
        jax.lax.fori_loop(0, (BW * H + L) // L, _rcp, None)

        def _norm(n, _):
            invrow = inv[pl.ds(n * 8, L)]
            ybase = n * (H * FG)
            for h in range(H):
                ch = _vgather(invrow, jnp.full((L,), h, jnp.int32))
                for k in range(FG // L):
                    sl = pl.ds(ybase + h * FG + k * L, L)
                    ylocal[sl] = ylocal[sl] * ch
            return None
        jax.lax.fori_loop(0, BW, _norm, None)

        pltpu.sync_copy(ylocal, y_hbm.at[pl.ds(lo * (H * FG), BW * H * FG)])


def _k2(ab, x_g, src, dst, mvec16):
    mesh = plsc.VectorSubcoreMesh(core_axis_name="c", subcore_axis_name="s")
    f = pl.kernel(
        _k2_body,
        out_type=jax.ShapeDtypeStruct((NP * H * FG,), jnp.float32),
        mesh=mesh,
        compiler_params=pltpu.CompilerParams(needs_layout_passes=False,
                                             use_tc_tiling_on_sc=False),
        scratch_types=[
            pltpu.VMEM((CCAP + L,), jnp.int32),
            pltpu.VMEM((CCAP + L,), jnp.int32),
            pltpu.VMEM((BCAP + GB,), jnp.int32),
            pltpu.VMEM((BCAP + GB,), jnp.int32),
            pltpu.VMEM((GB, 2 * H), jnp.float32),
            pltpu.VMEM((GB, 2 * H), jnp.float32),
            pltpu.VMEM((GB, FG), jnp.float32),
            pltpu.VMEM((BW * H + L,), jnp.float32),
            pltpu.VMEM((BW * H + L,), jnp.float32),
            pltpu.VMEM((BW * H * FG,), jnp.float32),
            pltpu.VMEM((ECHUNK,), jnp.int32),
            pltpu.VMEM((ECHUNK,), jnp.int32),
            pltpu.VMEM((L,), jnp.float32),
            pltpu.SemaphoreType.DMA,
            pltpu.SemaphoreType.DMA,
            pltpu.SemaphoreType.DMA,
        ],
    )
    return f(ab, x_g, src, dst, mvec16)


def kernel(x, edge_index, batch, W, att_src, att_dst, bias_conv, fc_w, fc_b):
    x_pad = jnp.zeros((NP, FP), jnp.float32).at[:N, :F].set(x)
    w_pad = jnp.zeros((FP, H * C), jnp.float32).at[:F].set(W)
    ab, m = _k1(x_pad, w_pad, att_src[0], att_dst[0])

    x_g = x_pad[:N, :FG]
    mvec16 = jnp.broadcast_to(m.reshape(()), (16,))
    y_flat = _k2(ab[:N], x_g, edge_index[0], edge_index[1], mvec16)
    y_pad = y_flat.reshape(NP, H * FG)

    w_bd = jnp.zeros((H, FG, H, C), jnp.float32)
    for h in range(H):
        w_bd = w_bd.at[h, :F, h, :].set(W[:, h * C:(h + 1) * C])
    w_bd = w_bd.reshape(H * FG, H * C)

    batch2 = jnp.full((1, NP), G, jnp.int32).at[0, :N].set(batch.astype(jnp.int32))
    out = _k3(y_pad, w_bd, bias_conv.reshape(1, H * C), fc_w, batch2,
              fc_b.reshape(1, 1))
    return out


# speedup vs baseline: 284.6726x; 1.3976x over previous
import functools
import jax
import jax.numpy as jnp
from jax import lax
from jax.experimental import pallas as pl
from jax.experimental.pallas import tpu as pltpu

N, E, G = 10000, 320000, 256
F, H, C = 79, 8, 128
NP = 10240
FP = 128
FG = 80
BLK = 1024


def _k1_body(x_ref, w_ref, ats_ref, atd_ref, ab_ref, m_ref):
    w3 = w_ref[...].reshape(FP, H, C)
    ws = (w3 * ats_ref[...][None, :, :]).sum(-1)
    wd = (w3 * atd_ref[...][None, :, :]).sum(-1)
    x = x_ref[...]
    a_src = jax.lax.dot(x, ws, preferred_element_type=jnp.float32)
    a_dst = jax.lax.dot(x, wd, preferred_element_type=jnp.float32)
    ab_ref[...] = jnp.concatenate([a_src, a_dst], axis=1)
    m_ref[...] = (jnp.max(a_src) + jnp.max(a_dst)).reshape(1, 1)


def _k1(x_pad, w_pad, att_src, att_dst):
    return pl.pallas_call(
        _k1_body,
        out_shape=[
            jax.ShapeDtypeStruct((NP, 2 * H), jnp.float32),
            jax.ShapeDtypeStruct((1, 1), jnp.float32),
        ],
    )(x_pad, w_pad, att_src, att_dst)


def _k3_body(y_ref, wbd_ref, bias_ref, fcw_ref, batch_ref, fcb_ref, out_ref,
             sums_ref, counts_ref):
    i = pl.program_id(0)

    @pl.when(i == 0)
    def _init():
        sums_ref[...] = jnp.zeros_like(sums_ref)
        counts_ref[...] = jnp.zeros_like(counts_ref)

    y = y_ref[...]
    o = jax.lax.dot(y, wbd_ref[...], preferred_element_type=jnp.float32)
    o = jnp.maximum(o + bias_ref[...], 0.0)
    z = jax.lax.dot(o, fcw_ref[...], preferred_element_type=jnp.float32)

    gids = jax.lax.broadcasted_iota(jnp.int32, (G, BLK), 0)
    oht = (gids == batch_ref[...]).astype(jnp.float32)
    sums_ref[...] += jax.lax.dot(oht, z, preferred_element_type=jnp.float32)
    counts_ref[...] += jnp.sum(oht, axis=1, keepdims=True)

    @pl.when(i == pl.num_programs(0) - 1)
    def _fin():
        out_ref[...] = (sums_ref[...] / jnp.maximum(counts_ref[...], 1.0)
                        + fcb_ref[...])


def _k3(y_pad, w_bd, bias2, fc_w, batch2, fcb2):
    nblk = NP // BLK
    return pl.pallas_call(
        _k3_body,
        grid=(nblk,),
        in_specs=[
            pl.BlockSpec((BLK, H * FG), lambda i: (i, 0)),
            pl.BlockSpec((H * FG, H * C), lambda i: (0, 0)),
            pl.BlockSpec((1, H * C), lambda i: (0, 0)),
            pl.BlockSpec((H * C, 1), lambda i: (0, 0)),
            pl.BlockSpec((1, BLK), lambda i: (0, i)),
            pl.BlockSpec((1, 1), lambda i: (0, 0)),
        ],
        out_specs=pl.BlockSpec((G, 1), lambda i: (0, 0)),
        out_shape=jax.ShapeDtypeStruct((G, 1), jnp.float32),
        scratch_shapes=[
            pltpu.VMEM((G, 1), jnp.float32),
            pltpu.VMEM((G, 1), jnp.float32),
        ],
    )(y_pad, w_bd, bias2, fc_w, batch2, fcb2)


from jax.experimental.pallas import tpu_sc as plsc

NC, NS, L = 2, 16, 16
NW = NC * NS
BW = 64
NB = NP // BW
ROUNDS = NB // NW
CCAP = 11264
BCAP = 2432
ECHUNK = 1280
GB = 64


def _vgather(vec, idx):
    dnums = jax.lax.GatherDimensionNumbers(
        offset_dims=(), collapsed_slice_dims=(0,), start_index_map=(0,))
    return jax.lax.gather(vec, idx[:, None], dnums, (1,),
                          mode=jax.lax.GatherScatterMode.PROMISE_IN_BOUNDS)


def _k2_body(ab_hbm, x_hbm, src_hbm, dst_hbm, mv_hbm, y_hbm,
             csl, cdl, slist, dlist, absrc0, abdst0, xbuf0,
             absrc1, abdst1, xbuf1, denom, inv, ylocal,
             srcc0, dstc0, srcc1, dstc1, mvec, semA, semB):
    wid = jax.lax.axis_index("s") * NC + jax.lax.axis_index("c")
    iota = jax.lax.iota(jnp.int32, L)
    lane8 = iota < 8
    rot8 = 8 + jnp.bitwise_and(iota, 7)
    zeros16 = jnp.zeros((L,), jnp.float32)

    pltpu.sync_copy(mv_hbm, mvec)
    mval = mvec[...]

    def _ch_issue(c, sbuf, dbuf, sem):
        pltpu.async_copy(src_hbm.at[pl.ds(c * ECHUNK, ECHUNK)], sbuf, sem)
        pltpu.async_copy(dst_hbm.at[pl.ds(c * ECHUNK, ECHUNK)], dbuf, sem)

    def _ch_wait(c, sbuf, dbuf, sem):
        pltpu.make_async_copy(
            src_hbm.at[pl.ds(c * ECHUNK, ECHUNK)], sbuf, sem).wait()
        pltpu.make_async_copy(
            dst_hbm.at[pl.ds(c * ECHUNK, ECHUNK)], dbuf, sem).wait()

    def _ch_proc(cnt, sbuf, dbuf):
        def _grp(j, cnt):
            s16 = sbuf[pl.ds(j * L, L)]
            d16 = dbuf[pl.ds(j * L, L)]
            bkt = jax.lax.shift_right_logical(d16, 6)
            msk = jnp.bitwise_and(bkt, NW - 1) == wid
            run = plsc.cumsum(jnp.where(msk, 1, 0))
            pos = cnt + run - 1
            plsc.store_scatter(csl, [pos], s16, mask=msk)
            plsc.store_scatter(cdl, [pos], d16, mask=msk)
            return jnp.minimum(cnt + run[L - 1], CCAP)

        return jax.lax.fori_loop(0, ECHUNK // L, _grp, cnt)

    NCH = E // ECHUNK

    _ch_issue(0, srcc0, dstc0, semA)

    def _pair(g, cnt):
        _ch_issue(2 * g + 1, srcc1, dstc1, semB)
        _ch_wait(2 * g, srcc0, dstc0, semA)
        cnt = _ch_proc(cnt, srcc0, dstc0)

        @pl.when(2 * g + 2 < NCH)
        def _():
            _ch_issue(2 * g + 2, srcc0, dstc0, semA)

        _ch_wait(2 * g + 1, srcc1, dstc1, semB)
        cnt = _ch_proc(cnt, srcc1, dstc1)
        return cnt

    cnt_all = jax.lax.fori_loop(0, NCH // 2, _pair, 0)
    nit_all = (cnt_all + L - 1) // L

    for r in range(ROUNDS):
        bucket = r * NW + wid
        lo = bucket * BW

        def _zy(i, _):
            ylocal[pl.ds(i * L, L)] = zeros16
            return None
        jax.lax.fori_loop(0, BW * H * FG // L, _zy, None)

        def _zd(i, _):
            denom[pl.ds(i * L, L)] = zeros16
            return None
        jax.lax.fori_loop(0, (BW * H + L) // L, _zd, None)

        def _zi(i, _):
            slist[pl.ds(i * L, L)] = jnp.zeros((L,), jnp.int32)
            dlist[pl.ds(i * L, L)] = jnp.zeros((L,), jnp.int32)
            return None
        jax.lax.fori_loop(0, (BCAP + GB) // L, _zi, None)

        def _part(i, cnt):
            s16 = csl[pl.ds(i * L, L)]
            d16 = cdl[pl.ds(i * L, L)]
            valid = (i * L + iota) < cnt_all
            msk = valid & (d16 >= lo) & (d16 < lo + BW)
            run = plsc.cumsum(jnp.where(msk, 1, 0))
            pos = cnt + run - 1
            plsc.store_scatter(slist, [pos], s16, mask=msk)
            plsc.store_scatter(dlist, [pos], d16, mask=msk)
            return jnp.minimum(cnt + run[L - 1], BCAP)

        cnt = jax.lax.fori_loop(0, nit_all, _part, 0)
        nblk = (cnt + GB - 1) // GB

        def _bk_issue(k, bs, bd, bx, sem):
            idx_s = slist.at[pl.ds(k * GB, GB)]
            idx_d = dlist.at[pl.ds(k * GB, GB)]
            pltpu.async_copy(ab_hbm.at[idx_s], bs, sem)
            pltpu.async_copy(ab_hbm.at[idx_d], bd, sem)
            pltpu.async_copy(x_hbm.at[idx_s], bx, sem)

        def _bk_wait(k, bs, bd, bx, sem):
            idx_s = slist.at[pl.ds(k * GB, GB)]
            idx_d = dlist.at[pl.ds(k * GB, GB)]
            pltpu.make_async_copy(ab_hbm.at[idx_s], bs, sem).wait()
            pltpu.make_async_copy(ab_hbm.at[idx_d], bd, sem).wait()
            pltpu.make_async_copy(x_hbm.at[idx_s], bx, sem).wait()

        def _bk_proc(b, bs, bd, bx):
            def _edge(j, v):
                @pl.when(b * GB + j < cnt)
                def _():
                    rs = bs[j, :]
                    rd = _vgather(bd[j, :], rot8)
                    al = rs + rd
                    al = jnp.maximum(al, 0.2 * al)
                    eal = jnp.exp(al - mval)
                    dstloc = dlist[pl.ds(b * GB + j, L)][0] - lo
                    plsc.addupdate_scatter(
                        denom, [dstloc * 8 + iota], eal, mask=lane8)
                    xs = [bx[j, pl.ds(k * L, L)] for k in range(FG // L)]
                    ybase = dstloc * (H * FG)
                    for h in range(H):
                        ch = _vgather(eal, jnp.full((L,), h, jnp.int32))
                        for k in range(FG // L):
                            plsc.addupdate(
                                ylocal.at[pl.ds(ybase + h * FG + k * L, L)],
                                ch * xs[k])
                return v
            jax.lax.fori_loop(0, GB, _edge, None)

        @pl.when(nblk > 0)
        def _():
            _bk_issue(0, absrc0, abdst0, xbuf0, semA)

        def _swp(g, _):
            @pl.when(2 * g + 1 < nblk)
            def _():
                _bk_issue(2 * g + 1, absrc1, abdst1, xbuf1, semB)

            _bk_wait(2 * g, absrc0, abdst0, xbuf0, semA)
            _bk_proc(2 * g, absrc0, abdst0, xbuf0)

            @pl.when(2 * g + 2 < nblk)
            def _():
                _bk_issue(2 * g + 2, absrc0, abdst0, xbuf0, semA)

            @pl.when(2 * g + 1 < nblk)
            def _():
                _bk_wait(2 * g + 1, absrc1, abdst1, xbuf1, semB)
                _bk_proc(2 * g + 1, absrc1, abdst1, xbuf1)
            return None

        jax.lax.fori_loop(0, (nblk + 1) // 2, _swp, None)

        def _rcp(i, _):
            inv[pl.ds(i * L, L)] = 1.0 / (denom[pl.ds(i * L, L)] + 1e-16)
            return None
        jax.lax.fori_loop(0, (BW * H + L) // L, _rcp, None)

        def _norm(n, _):
            invrow = inv[pl.ds(n * 8, L)]
            ybase = n * (H * FG)
            for h in range(H):
                ch = _vgather(invrow, jnp.full((L,), h, jnp.int32))
                for k in range(FG // L):
                    sl = pl.ds(ybase + h * FG + k * L, L)
                    ylocal[sl] = ylocal[sl] * ch
            return None
        jax.lax.fori_loop(0, BW, _norm, None)

        pltpu.sync_copy(ylocal, y_hbm.at[pl.ds(lo * (H * FG), BW * H * FG)])


def _k2(ab, x_g, src, dst, mvec16):
    mesh = plsc.VectorSubcoreMesh(core_axis_name="c", subcore_axis_name="s")
    f = pl.kernel(
        _k2_body,
        out_type=jax.ShapeDtypeStruct((NP * H * FG,), jnp.float32),
        mesh=mesh,
        compiler_params=pltpu.CompilerParams(needs_layout_passes=False,
                                             use_tc_tiling_on_sc=False),
        scratch_types=[
            pltpu.VMEM((CCAP + L,), jnp.int32),
            pltpu.VMEM((CCAP + L,), jnp.int32),
            pltpu.VMEM((BCAP + GB,), jnp.int32),
            pltpu.VMEM((BCAP + GB,), jnp.int32),
            pltpu.VMEM((GB, 2 * H), jnp.float32),
            pltpu.VMEM((GB, 2 * H), jnp.float32),
            pltpu.VMEM((GB, FG), jnp.float32),
            pltpu.VMEM((GB, 2 * H), jnp.float32),
            pltpu.VMEM((GB, 2 * H), jnp.float32),
            pltpu.VMEM((GB, FG), jnp.float32),
            pltpu.VMEM((BW * H + L,), jnp.float32),
            pltpu.VMEM((BW * H + L,), jnp.float32),
            pltpu.VMEM((BW * H * FG,), jnp.float32),
            pltpu.VMEM((ECHUNK,), jnp.int32),
            pltpu.VMEM((ECHUNK,), jnp.int32),
            pltpu.VMEM((ECHUNK,), jnp.int32),
            pltpu.VMEM((ECHUNK,), jnp.int32),
            pltpu.VMEM((L,), jnp.float32),
            pltpu.SemaphoreType.DMA,
            pltpu.SemaphoreType.DMA,
        ],
    )
    return f(ab, x_g, src, dst, mvec16)


def kernel(x, edge_index, batch, W, att_src, att_dst, bias_conv, fc_w, fc_b):
    x_pad = jnp.zeros((NP, FP), jnp.float32).at[:N, :F].set(x)
    w_pad = jnp.zeros((FP, H * C), jnp.float32).at[:F].set(W)
    ab, m = _k1(x_pad, w_pad, att_src[0], att_dst[0])

    x_g = x_pad[:N, :FG]
    mvec16 = jnp.broadcast_to(m.reshape(()), (16,))
    y_flat = _k2(ab[:N], x_g, edge_index[0], edge_index[1], mvec16)
    y_pad = y_flat.reshape(NP, H * FG)

    w_bd = jnp.zeros((H, FG, H, C), jnp.float32)
    for h in range(H):
        w_bd = w_bd.at[h, :F, h, :].set(W[:, h * C:(h + 1) * C])
    w_bd = w_bd.reshape(H * FG, H * C)

    batch2 = jnp.full((1, NP), G, jnp.int32).at[0, :N].set(batch.astype(jnp.int32))
    out = _k3(y_pad, w_bd, bias_conv.reshape(1, H * C), fc_w, batch2,
              fc_b.reshape(1, 1))
    return out
